# baseline (speedup 1.0000x reference)
import functools

import jax
import jax.numpy as jnp
from jax import lax
from jax.experimental import pallas as pl
from jax.experimental.pallas import tpu as pltpu
from jax.experimental.pallas import tpu_sc as plsc

B = 16384
D = 2049
DM = 2048
L = 16
DPAD = 2176
NC, NS = 2, 16
NW = NC * NS
RPW = B // NW
CH_ROWS = 16
NCH = RPW // CH_ROWS
NSL = DM // L


def _sc_body(inp, idxa, out2, table, tail, y_out,
             buf0, buf1, embv, out2v, tailv, yv, scr, idxv, sem0, sem1):
    cid = lax.axis_index("c")
    sid = lax.axis_index("s")
    wid = sid * NC + cid
    row0 = wid * RPW

    pltpu.sync_copy(idxa, idxv)
    pltpu.sync_copy(out2.at[pl.ds(row0, RPW)], out2v)
    pltpu.sync_copy(tail.at[pl.ds(row0, RPW)], tailv)
    pltpu.async_copy(table.at[idxv], embv, sem0).wait()

    zero = jnp.zeros((L,), jnp.float32)
    lane = lax.broadcasted_iota(jnp.int32, (L,), 0)
    scr_off = lane * (L + 1)
    emb_t = plsc.load_gather(embv, [jnp.zeros((L,), jnp.int32),
                                    jnp.full((L,), DM, jnp.int32)])

    bufs = (buf0, buf1)
    sems = (sem0, sem1)

    for b in range(2):
        pltpu.async_copy(
            inp.at[pl.ds(row0 + b * CH_ROWS, CH_ROWS), pl.ds(0, DM)],
            bufs[b], sems[b])

    def pair(g, carry):
        for b in range(2):
            ch = 2 * g + b
            buf = bufs[b]
            sem = sems[b]
            r_base = row0 + ch * CH_ROWS
            pltpu.make_async_copy(
                inp.at[pl.ds(r_base, CH_ROWS), pl.ds(0, DM)], buf, sem).wait()

            def cstep(c, accs):
                es = embv[0, pl.ds(c * L, L)]
                return tuple(accs[r] + buf[r, pl.ds(c * L, L)] * es
                             for r in range(CH_ROWS))

            accs = lax.fori_loop(0, NSL, cstep,
                                 tuple(zero for _ in range(CH_ROWS)),
                                 unroll=2)

            for r in range(CH_ROWS):
                scr[pl.ds(r * (L + 1), L)] = accs[r]
            rowsum = zero
            for l in range(L):
                rowsum = rowsum + plsc.load_gather(scr, [scr_off + l])

            rowsum = rowsum + tailv[pl.ds(ch * CH_ROWS, CH_ROWS)] * emb_t
            y = jnp.minimum(rowsum, out2v[pl.ds(ch * CH_ROWS, CH_ROWS)])
            yv[pl.ds(ch * CH_ROWS, CH_ROWS)] = y

            @pl.when(ch + 2 < NCH)
            def _():
                pltpu.async_copy(
                    inp.at[pl.ds(r_base + 2 * CH_ROWS, CH_ROWS),
                           pl.ds(0, DM)],
                    buf, sem)
        return carry

    lax.fori_loop(0, NCH // 2, pair, 0)
    pltpu.sync_copy(yv, y_out.at[pl.ds(row0, RPW)])


_sc_call = pl.kernel(
    _sc_body,
    out_type=jax.ShapeDtypeStruct((B,), jnp.float32),
    mesh=plsc.VectorSubcoreMesh(core_axis_name="c", subcore_axis_name="s",
                                num_cores=NC, num_subcores=NS),
    scratch_types=[
        pltpu.VMEM((CH_ROWS, DM), jnp.float32),
        pltpu.VMEM((CH_ROWS, DM), jnp.float32),
        pltpu.VMEM((1, DPAD), jnp.float32),
        pltpu.VMEM((RPW,), jnp.float32),
        pltpu.VMEM((RPW,), jnp.float32),
        pltpu.VMEM((RPW,), jnp.float32),
        pltpu.VMEM(((L + 1) * L,), jnp.float32),
        pltpu.VMEM((1,), jnp.int32),
        pltpu.SemaphoreType.DMA,
        pltpu.SemaphoreType.DMA,
    ],
    compiler_params=pltpu.CompilerParams(needs_layout_passes=False,
                                         use_tc_tiling_on_sc=True),
)


def _max_body(y_ref, o_ref):
    o_ref[0, 0] = jnp.max(y_ref[...])


def _final_max(y):
    return pl.pallas_call(
        _max_body,
        out_shape=jax.ShapeDtypeStruct((1, 1), jnp.float32),
        out_specs=pl.BlockSpec(memory_space=pltpu.SMEM),
    )(y.reshape(B // 128, 128))


def kernel(input, idx, out2, emb_table):
    idxa = jnp.full((1,), idx, jnp.int32)
    table = jnp.pad(emb_table, ((0, 0), (0, DPAD - D)))
    tail = input[:, DM]
    y = _sc_call(input, idxa, out2, table, tail)
    out = _final_max(y).reshape(1)
    return (out, y.reshape(1, B))


# speedup vs baseline: 2.4966x; 2.4966x over previous
import functools

import jax
import jax.numpy as jnp
from jax import lax
from jax.experimental import pallas as pl
from jax.experimental.pallas import tpu as pltpu
from jax.experimental.pallas import tpu_sc as plsc

B = 16384
D = 2049
DM = 2048
L = 16
DPAD = 2176
NC, NS = 2, 16
NW = NC * NS
BPW = B // NW
NJ = BPW // L
KCH = 64
NCH = DM // KCH


def _sc_body(inp_t, idxa, out2, table, y_out,
             buf0, buf1, embv, out2v, yv, tlv, idxv, sem0, sem1):
    cid = lax.axis_index("c")
    sid = lax.axis_index("s")
    wid = sid * NC + cid
    b0 = wid * BPW

    pltpu.sync_copy(idxa, idxv)
    pltpu.sync_copy(out2.at[pl.ds(b0, BPW)], out2v)
    pltpu.sync_copy(inp_t.at[pl.ds(DM, 1), pl.ds(b0, BPW)], tlv)
    pltpu.async_copy(table.at[idxv], embv, sem0).wait()

    zero = jnp.zeros((L,), jnp.float32)
    z16 = jnp.zeros((L,), jnp.int32)
    emb_t = plsc.load_gather(embv, [z16, jnp.full((L,), DM, jnp.int32)])

    bufs = (buf0, buf1)
    sems = (sem0, sem1)

    for b in range(2):
        pltpu.async_copy(
            inp_t.at[pl.ds(b * KCH, KCH), pl.ds(b0, BPW)], bufs[b], sems[b])

    def pair(g, accs):
        for b in range(2):
            ch = 2 * g + b
            buf = bufs[b]
            sem = sems[b]
            k0 = ch * KCH
            pltpu.make_async_copy(
                inp_t.at[pl.ds(k0, KCH), pl.ds(b0, BPW)], buf, sem).wait()

            new = []
            for half in range(2):
                sub = accs[half * (NJ // 2):(half + 1) * (NJ // 2)]

                def kstep(kk, a, _half=half):
                    ebk = plsc.load_gather(
                        embv, [z16, z16 + (k0 + kk)])
                    off = _half * (NJ // 2) * L
                    return tuple(
                        a[j] + buf[kk, pl.ds(off + j * L, L)] * ebk
                        for j in range(NJ // 2))

                new.extend(lax.fori_loop(0, KCH, kstep, tuple(sub)))
            accs = tuple(new)

            @pl.when(ch + 2 < NCH)
            def _():
                pltpu.async_copy(
                    inp_t.at[pl.ds(k0 + 2 * KCH, KCH), pl.ds(b0, BPW)],
                    buf, sem)
        return accs

    accs = lax.fori_loop(0, NCH // 2, pair,
                         tuple(zero for _ in range(NJ)))

    for j in range(NJ):
        a = accs[j] + tlv[0, pl.ds(j * L, L)] * emb_t
        yv[pl.ds(j * L, L)] = jnp.minimum(a, out2v[pl.ds(j * L, L)])
    pltpu.sync_copy(yv, y_out.at[pl.ds(b0, BPW)])


_sc_call = pl.kernel(
    _sc_body,
    out_type=jax.ShapeDtypeStruct((B,), jnp.float32),
    mesh=plsc.VectorSubcoreMesh(core_axis_name="c", subcore_axis_name="s",
                                num_cores=NC, num_subcores=NS),
    scratch_types=[
        pltpu.VMEM((KCH, BPW), jnp.float32),
        pltpu.VMEM((KCH, BPW), jnp.float32),
        pltpu.VMEM((1, DPAD), jnp.float32),
        pltpu.VMEM((BPW,), jnp.float32),
        pltpu.VMEM((BPW,), jnp.float32),
        pltpu.VMEM((1, BPW), jnp.float32),
        pltpu.VMEM((1,), jnp.int32),
        pltpu.SemaphoreType.DMA,
        pltpu.SemaphoreType.DMA,
    ],
    compiler_params=pltpu.CompilerParams(needs_layout_passes=False),
)


def _max_body(y_ref, o_ref):
    o_ref[0, 0] = jnp.max(y_ref[...])


def _final_max(y):
    return pl.pallas_call(
        _max_body,
        out_shape=jax.ShapeDtypeStruct((1, 1), jnp.float32),
        out_specs=pl.BlockSpec(memory_space=pltpu.SMEM),
    )(y.reshape(B // 128, 128))


def kernel(input, idx, out2, emb_table):
    idxa = jnp.full((1,), idx, jnp.int32)
    table = jnp.pad(emb_table, ((0, 0), (0, DPAD - D)))
    y = _sc_call(input.T, idxa, out2, table)
    out = _final_max(y).reshape(1)
    return (out, y.reshape(1, B))
